# Initial kernel scaffold; baseline (speedup 1.0000x reference)
import jax
import jax.numpy as jnp
from jax.experimental import pallas as pl


def kernel(node_feats, edge_feats, scalar_edge_feats, lengths, senders, receivers, W1, W2):
    raise NotImplementedError("write your pallas kernel here")


# speedup vs baseline: 1.7281x; 1.7281x over previous
import math

import jax
import jax.numpy as jnp
import numpy as np
from jax import lax
from jax.experimental import pallas as pl
from jax.experimental.pallas import tpu as pltpu
from jax.experimental.pallas import tpu_sc as plsc

N_NODES = 50000
N_EDGES = 800000
S0 = 32
V0 = 16
MLP_IN = 73
HID = 64
AVG_NUM_NEIGHBORS = 16.0

NC = 2
NS = 16
NW = NC * NS

RROWS = 784
NRANGE = 64
NPAD = RROWS * NRANGE
ACC_ROWS = 800
BCAP = 1088
_INV = 1.0 / 784.0


_GK = 200
_EPW = N_EDGES // NW


def _gather_body(nf_hbm, snd_hbm, rcv_hbm, out_hbm,
                 sidx, ridx, srow, rrow, sem_s, sem_r):
    wid = lax.axis_index("s") * NC + lax.axis_index("c")
    base0 = wid * _EPW

    def step(i, _):
        base = base0 + i * _GK
        pltpu.sync_copy(snd_hbm.at[pl.ds(base, _GK)], sidx)
        pltpu.sync_copy(rcv_hbm.at[pl.ds(base, _GK)], ridx)
        c1 = pltpu.async_copy(nf_hbm.at[sidx], srow, sem_s)
        c2 = pltpu.async_copy(nf_hbm.at[ridx], rrow, sem_r)
        c1.wait()
        c2.wait()

        def pack(j, _):
            srow[j, pl.ds(80, 16)] = rrow[j, pl.ds(0, 16)]
            srow[j, pl.ds(96, 16)] = rrow[j, pl.ds(16, 16)]
            return 0

        lax.fori_loop(0, _GK, pack, 0)
        pltpu.sync_copy(srow, out_hbm.at[pl.ds(base, _GK)])
        return 0

    lax.fori_loop(0, _EPW // _GK, step, 0)


_gather = pl.kernel(
    _gather_body,
    out_type=jax.ShapeDtypeStruct((N_EDGES, 128), jnp.float32),
    mesh=plsc.VectorSubcoreMesh(core_axis_name="c", subcore_axis_name="s"),
    scratch_types=[
        pltpu.VMEM((_GK,), jnp.int32),
        pltpu.VMEM((_GK,), jnp.int32),
        pltpu.VMEM((_GK, 128), jnp.float32),
        pltpu.VMEM((_GK, 128), jnp.float32),
        pltpu.SemaphoreType.DMA,
        pltpu.SemaphoreType.DMA,
    ],
)


_EB = 1000

_T48 = np.zeros((3, 48), np.float32)
_T48[np.arange(48) % 3, np.arange(48)] = 1.0
_P48 = np.zeros((48, 16), np.float32)
_P48[np.arange(48), np.arange(48) // 3] = 1.0
_T96 = np.zeros((3, 96), np.float32)
_T96[np.arange(96) % 3, np.arange(96)] = 1.0
_R96 = np.zeros((32, 96), np.float32)
_R96[np.arange(96) // 3, np.arange(96)] = 1.0


def _edge_body(gth, ef, sef, ln, w1, w2, t48, p48, t96, r96, out):
    ms = gth[:, :S0]
    sv = gth[:, S0:S0 + 3 * V0]
    rs = gth[:, 80:112]
    ev = ef[:, 1:4]
    x = jnp.concatenate([ms, rs, sef[...], ln[...]], axis=1)
    h = jnp.dot(x, w1[...], preferred_element_type=jnp.float32,
                 precision=lax.Precision.HIGHEST)
    h = h * (1.0 / math.sqrt(float(MLP_IN)))
    h = h / (1.0 + jnp.exp(-h))
    mix = jnp.dot(h, w2[...], preferred_element_type=jnp.float32,
                 precision=lax.Precision.HIGHEST)
    mix = mix * (1.0 / math.sqrt(float(HID)))
    ev48 = jnp.dot(ev, t48[...], preferred_element_type=jnp.float32,
                 precision=lax.Precision.HIGHEST)
    o0 = jnp.dot(sv * ev48, p48[...], preferred_element_type=jnp.float32,
                 precision=lax.Precision.HIGHEST)
    o0 = o0 * mix[:, :V0] * (1.0 / (math.sqrt(3.0) * AVG_NUM_NEIGHBORS))
    ev96 = jnp.dot(ev, t96[...], preferred_element_type=jnp.float32,
                 precision=lax.Precision.HIGHEST)
    a = ms * mix[:, V0:]
    o1 = jnp.dot(a, r96[...], preferred_element_type=jnp.float32,
                 precision=lax.Precision.HIGHEST)
    o1 = o1 * ev96 * (1.0 / AVG_NUM_NEIGHBORS)
    out[...] = jnp.concatenate(
        [o0, o1, jnp.zeros((_EB, 16), jnp.float32)], axis=1)


def _edge_compute(gth, ef, sef, ln, w1, w2):
    nb = N_EDGES // _EB
    full = lambda i: (0, 0)
    return pl.pallas_call(
        _edge_body,
        grid=(nb,),
        in_specs=[
            pl.BlockSpec((_EB, 128), lambda i: (i, 0)),
            pl.BlockSpec((_EB, 4), lambda i: (i, 0)),
            pl.BlockSpec((_EB, 8), lambda i: (i, 0)),
            pl.BlockSpec((_EB, 1), lambda i: (i, 0)),
            pl.BlockSpec((MLP_IN, HID), full),
            pl.BlockSpec((HID, 48), full),
            pl.BlockSpec((3, 48), full),
            pl.BlockSpec((48, 16), full),
            pl.BlockSpec((3, 96), full),
            pl.BlockSpec((32, 96), full),
        ],
        out_specs=pl.BlockSpec((_EB, 128), lambda i: (i, 0)),
        out_shape=jax.ShapeDtypeStruct((N_EDGES, 128), jnp.float32),
        compiler_params=pltpu.CompilerParams(
            dimension_semantics=("arbitrary",),
        ),
    )(gth, ef, sef, ln, w1, w2,
      jnp.asarray(_T48), jnp.asarray(_P48), jnp.asarray(_T96), jnp.asarray(_R96))


_BW = 1000
_SCAP = 3920


def _bin_body(rcv_hbm, bins_hbm, pcnt_hbm, win, st_e, st_r, st2, cntv, sem):
    wid = lax.axis_index("s") * NC + lax.axis_index("c")
    base0 = wid * _EPW
    iota = lax.iota(jnp.int32, 16)
    inv = jnp.float32(_INV)

    def window_step(i, curs):
        base = base0 + i * _BW
        pltpu.sync_copy(rcv_hbm.at[pl.ds(base, _BW)], win)
        new_curs = []
        for c in range(8):
            def vstep(j, cur, c=c):
                off = jnp.where(j < 62, j * 16, 984)
                valid = (j < 62) | (iota >= 8)
                r = win[pl.ds(off, 16)]
                rng = ((r.astype(jnp.float32) + 0.5) * inv).astype(jnp.int32)
                mask = valid & (lax.shift_right_logical(rng, 3) == c)
                eid = base + off + iota
                cum = plsc.cumsum(mask.astype(jnp.int32))
                pos = cur + cum - mask.astype(jnp.int32)
                crow = jnp.full((16,), c, jnp.int32)
                plsc.store_scatter(st_r, [crow, pos], r, mask=mask)
                plsc.store_scatter(st_e, [crow, pos], eid, mask=mask)
                return cur + jnp.max(cum)

            new_curs.append(lax.fori_loop(0, 63, vstep, curs[c]))
        return tuple(new_curs)

    curs = lax.fori_loop(0, _EPW // _BW, window_step, (0,) * 8)

    for c in range(8):
        cnt_c = curs[c]
        trips = lax.shift_right_logical(cnt_c + 15, 4)
        cvec = jnp.zeros((16,), jnp.int32)
        for f in range(8):
            rng_t = c * 8 + f

            def vstep(j, cur, c=c, rng_t=rng_t):
                valid = (j * 16 + iota) < cnt_c
                r = st_r[c, pl.ds(j * 16, 16)]
                e = st_e[c, pl.ds(j * 16, 16)]
                rng = ((r.astype(jnp.float32) + 0.5) * inv).astype(jnp.int32)
                mask = valid & (rng == rng_t)
                local = r - rng * RROWS
                packed = e | lax.shift_left(local, 20)
                cum = plsc.cumsum(mask.astype(jnp.int32))
                pos = cur + cum - mask.astype(jnp.int32)
                plsc.store_scatter(st2, [pos], packed, mask=mask)
                return cur + jnp.max(cum)

            cnt_f = lax.fori_loop(0, trips, vstep, 0)
            npad = (-cnt_f) & 63
            fill = (base0 + iota) | lax.shift_left(RROWS + iota, 20)
            for q in range(4):
                mask = iota < (npad - q * 16)
                plsc.store_scatter(st2, [cnt_f + q * 16 + iota], fill,
                                   mask=mask)
            padded = cnt_f + npad
            cvec = jnp.where(iota == f, padded, cvec)
            pltpu.sync_copy(
                st2, bins_hbm.at[pl.ds((rng_t * NW + wid) * BCAP, BCAP)])
        cntv[...] = cvec
        pltpu.sync_copy(cntv, pcnt_hbm.at[pl.ds((wid * 8 + c) * 16, 16)])


_bin = pl.kernel(
    _bin_body,
    out_type=(
        jax.ShapeDtypeStruct((NRANGE * NW * BCAP,), jnp.int32),
        jax.ShapeDtypeStruct((NW * 8 * 16,), jnp.int32),
    ),
    mesh=plsc.VectorSubcoreMesh(core_axis_name="c", subcore_axis_name="s"),
    scratch_types=[
        pltpu.VMEM((_BW,), jnp.int32),
        pltpu.VMEM((8, _SCAP), jnp.int32),
        pltpu.VMEM((8, _SCAP), jnp.int32),
        pltpu.VMEM((BCAP,), jnp.int32),
        pltpu.VMEM((16,), jnp.int32),
        pltpu.SemaphoreType.DMA,
    ],
    compiler_params=pltpu.CompilerParams(needs_layout_passes=False),
)


_WIN = 64


def _scatter_body(msg_hbm, bins_hbm, pcnt_hbm, out_hbm,
                  ids, lbuf, cntv, rows, acc, sem):
    wid = lax.axis_index("s") * NC + lax.axis_index("c")
    iota = lax.iota(jnp.int32, 16)

    for q in range(2):
        rng = wid * 2 + q
        c = lax.shift_right_logical(rng, 3)
        f = rng & 7

        def zro(j, _):
            for ch in range(8):
                acc[j, pl.ds(ch * 16, 16)] = jnp.zeros((16,), jnp.float32)
            return 0

        lax.fori_loop(0, ACC_ROWS, zro, 0)

        def wloop(w, _):
            pltpu.sync_copy(pcnt_hbm.at[pl.ds((w * 8 + c) * 16, 16)], cntv)
            cnt = jnp.max(jnp.where(iota == f, cntv[...], 0))
            nwin = lax.shift_right_logical(cnt, 6)
            base = (rng * NW + w) * BCAP

            def wina(j, _):
                pltpu.sync_copy(bins_hbm.at[pl.ds(base + j * _WIN, _WIN)],
                                ids)
                for v in range(4):
                    pk = ids[pl.ds(v * 16, 16)]
                    lbuf[pl.ds(v * 16, 16)] = lax.shift_right_logical(pk, 20)
                    ids[pl.ds(v * 16, 16)] = pk & 0xFFFFF
                pltpu.async_copy(msg_hbm.at[ids], rows, sem).wait()

                def edge(e, _):
                    l16 = lbuf[pl.ds((e // 16) * 16, 16)]
                    loc = jnp.take(l16, jnp.full((16,), e % 16, jnp.int32))
                    for ch in range(8):
                        vals = rows[e, pl.ds(ch * 16, 16)]
                        plsc.addupdate_scatter(acc, [loc, ch * 16 + iota],
                                               vals)
                    return 0

                lax.fori_loop(0, _WIN, edge, 0)
                return 0

            lax.fori_loop(0, nwin, wina, 0)
            return 0

        lax.fori_loop(0, NW, wloop, 0)
        pltpu.sync_copy(acc, out_hbm.at[pl.ds(rng * ACC_ROWS, ACC_ROWS)])


_scatter = pl.kernel(
    _scatter_body,
    out_type=jax.ShapeDtypeStruct((NRANGE * ACC_ROWS, 128), jnp.float32),
    mesh=plsc.VectorSubcoreMesh(core_axis_name="c", subcore_axis_name="s"),
    scratch_types=[
        pltpu.VMEM((_WIN,), jnp.int32),
        pltpu.VMEM((_WIN,), jnp.int32),
        pltpu.VMEM((16,), jnp.int32),
        pltpu.VMEM((_WIN, 128), jnp.float32),
        pltpu.VMEM((ACC_ROWS, 128), jnp.float32),
        pltpu.SemaphoreType.DMA,
    ],
    compiler_params=pltpu.CompilerParams(needs_layout_passes=False),
)


@jax.jit
def kernel(node_feats, edge_feats, scalar_edge_feats, lengths, senders,
           receivers, W1, W2):
    nf128 = jnp.pad(node_feats, ((0, 0), (0, 48)))
    gth = _gather(nf128, senders, receivers)
    msg = _edge_compute(gth, edge_feats, scalar_edge_feats, lengths, W1, W2)
    bins, pcnt = _bin(receivers)
    out = _scatter(msg, bins, pcnt)
    out = out.reshape(NRANGE, ACC_ROWS, 128)[:, :RROWS, :]
    return out.reshape(NPAD, 128)[:N_NODES, :112]


# speedup vs baseline: 2.7716x; 1.6038x over previous
import math

import jax
import jax.numpy as jnp
import numpy as np
from jax import lax
from jax.experimental import pallas as pl
from jax.experimental.pallas import tpu as pltpu
from jax.experimental.pallas import tpu_sc as plsc

N_NODES = 50000
N_EDGES = 800000
S0 = 32
V0 = 16
MLP_IN = 73
HID = 64
AVG_NUM_NEIGHBORS = 16.0

NC = 2
NS = 16
NW = NC * NS

RROWS = 784
NRANGE = 64
NPAD = RROWS * NRANGE
ACC_ROWS = 800
BCAP = 1088
_INV = 1.0 / 784.0


_GK = 200
_EPW = N_EDGES // NW


def _gather_body(nf_hbm, snd_hbm, rcv_hbm, out_hbm,
                 sidx, ridx, srow, rrow, sem_s, sem_r):
    wid = lax.axis_index("s") * NC + lax.axis_index("c")
    base0 = wid * _EPW

    def step(i, _):
        base = base0 + i * _GK
        pltpu.sync_copy(snd_hbm.at[pl.ds(base, _GK)], sidx)
        pltpu.sync_copy(rcv_hbm.at[pl.ds(base, _GK)], ridx)
        c1 = pltpu.async_copy(nf_hbm.at[sidx], srow, sem_s)
        c2 = pltpu.async_copy(nf_hbm.at[ridx], rrow, sem_r)
        c1.wait()
        c2.wait()

        def pack(j, _):
            srow[j, pl.ds(80, 16)] = rrow[j, pl.ds(0, 16)]
            srow[j, pl.ds(96, 16)] = rrow[j, pl.ds(16, 16)]
            return 0

        lax.fori_loop(0, _GK, pack, 0)
        pltpu.sync_copy(srow, out_hbm.at[pl.ds(base, _GK)])
        return 0

    lax.fori_loop(0, _EPW // _GK, step, 0)


_gather = pl.kernel(
    _gather_body,
    out_type=jax.ShapeDtypeStruct((N_EDGES, 128), jnp.float32),
    mesh=plsc.VectorSubcoreMesh(core_axis_name="c", subcore_axis_name="s"),
    scratch_types=[
        pltpu.VMEM((_GK,), jnp.int32),
        pltpu.VMEM((_GK,), jnp.int32),
        pltpu.VMEM((_GK, 128), jnp.float32),
        pltpu.VMEM((_GK, 128), jnp.float32),
        pltpu.SemaphoreType.DMA,
        pltpu.SemaphoreType.DMA,
    ],
)


_EB = 4000

_T48 = np.zeros((3, 48), np.float32)
_T48[np.arange(48) % 3, np.arange(48)] = 1.0
_P48 = np.zeros((48, 16), np.float32)
_P48[np.arange(48), np.arange(48) // 3] = 1.0
_T96 = np.zeros((3, 96), np.float32)
_T96[np.arange(96) % 3, np.arange(96)] = 1.0
_R96 = np.zeros((32, 96), np.float32)
_R96[np.arange(96) // 3, np.arange(96)] = 1.0


def _split_dot(x, w):
    xh = x.astype(jnp.bfloat16)
    xl = (x - xh.astype(jnp.float32)).astype(jnp.bfloat16)
    wh = w.astype(jnp.bfloat16)
    wl = (w - wh.astype(jnp.float32)).astype(jnp.bfloat16)
    d = lambda a, b: jnp.dot(a, b, preferred_element_type=jnp.float32)
    return d(xh, wh) + (d(xh, wl) + d(xl, wh))


def _edge_body(gth, ef, sef, ln, w1, w2, p48, out):
    ms = gth[:, :S0]
    sv = gth[:, S0:S0 + 3 * V0]
    rs = gth[:, 80:112]
    ev = ef[:, 1:4]
    x = jnp.concatenate([ms, rs, sef[...], ln[...]], axis=1)
    h = _split_dot(x, w1[...]) * (1.0 / math.sqrt(float(MLP_IN)))
    h = h / (1.0 + jnp.exp(-h))
    mix = _split_dot(h, w2[...]) * (1.0 / math.sqrt(float(HID)))
    ev48 = jnp.concatenate([ev] * 16, axis=1)
    o0 = _split_dot(sv * ev48, p48[...])
    o0 = o0 * mix[:, :V0] * (1.0 / (math.sqrt(3.0) * AVG_NUM_NEIGHBORS))
    a = ms * mix[:, V0:] * (1.0 / AVG_NUM_NEIGHBORS)
    o1 = jnp.concatenate(
        [a * ef[:, 1:2], a * ef[:, 2:3], a * ef[:, 3:4]], axis=1)
    out[...] = jnp.concatenate(
        [o0, o1, jnp.zeros((_EB, 16), jnp.float32)], axis=1)


def _edge_compute(gth, ef, sef, ln, w1, w2):
    nb = N_EDGES // _EB
    full = lambda i: (0, 0)
    return pl.pallas_call(
        _edge_body,
        grid=(nb,),
        in_specs=[
            pl.BlockSpec((_EB, 128), lambda i: (i, 0)),
            pl.BlockSpec((_EB, 4), lambda i: (i, 0)),
            pl.BlockSpec((_EB, 8), lambda i: (i, 0)),
            pl.BlockSpec((_EB, 1), lambda i: (i, 0)),
            pl.BlockSpec((MLP_IN, HID), full),
            pl.BlockSpec((HID, 48), full),
            pl.BlockSpec((48, 16), full),
        ],
        out_specs=pl.BlockSpec((_EB, 128), lambda i: (i, 0)),
        out_shape=jax.ShapeDtypeStruct((N_EDGES, 128), jnp.float32),
        compiler_params=pltpu.CompilerParams(
            dimension_semantics=("arbitrary",),
        ),
    )(gth, ef, sef, ln, w1, w2, jnp.asarray(_P48))


_BW = 1000
_SCAP = 3920


def _bin_body(rcv_hbm, bins_hbm, pcnt_hbm, win, st_e, st_r, st2, cntv, sem):
    wid = lax.axis_index("s") * NC + lax.axis_index("c")
    base0 = wid * _EPW
    iota = lax.iota(jnp.int32, 16)
    inv = jnp.float32(_INV)

    def window_step(i, curs):
        base = base0 + i * _BW
        pltpu.sync_copy(rcv_hbm.at[pl.ds(base, _BW)], win)
        new_curs = []
        for c in range(8):
            def vstep(j, cur, c=c):
                off = jnp.where(j < 62, j * 16, 984)
                valid = (j < 62) | (iota >= 8)
                r = win[pl.ds(off, 16)]
                rng = ((r.astype(jnp.float32) + 0.5) * inv).astype(jnp.int32)
                mask = valid & (lax.shift_right_logical(rng, 3) == c)
                eid = base + off + iota
                cum = plsc.cumsum(mask.astype(jnp.int32))
                pos = cur + cum - mask.astype(jnp.int32)
                crow = jnp.full((16,), c, jnp.int32)
                plsc.store_scatter(st_r, [crow, pos], r, mask=mask)
                plsc.store_scatter(st_e, [crow, pos], eid, mask=mask)
                return cur + jnp.max(cum)

            new_curs.append(lax.fori_loop(0, 63, vstep, curs[c]))
        return tuple(new_curs)

    curs = lax.fori_loop(0, _EPW // _BW, window_step, (0,) * 8)

    for c in range(8):
        cnt_c = curs[c]
        trips = lax.shift_right_logical(cnt_c + 15, 4)
        cvec = jnp.zeros((16,), jnp.int32)
        for f in range(8):
            rng_t = c * 8 + f

            def vstep(j, cur, c=c, rng_t=rng_t):
                valid = (j * 16 + iota) < cnt_c
                r = st_r[c, pl.ds(j * 16, 16)]
                e = st_e[c, pl.ds(j * 16, 16)]
                rng = ((r.astype(jnp.float32) + 0.5) * inv).astype(jnp.int32)
                mask = valid & (rng == rng_t)
                local = r - rng * RROWS
                packed = e | lax.shift_left(local, 20)
                cum = plsc.cumsum(mask.astype(jnp.int32))
                pos = cur + cum - mask.astype(jnp.int32)
                plsc.store_scatter(st2, [pos], packed, mask=mask)
                return cur + jnp.max(cum)

            cnt_f = lax.fori_loop(0, trips, vstep, 0)
            npad = (-cnt_f) & 63
            fill = (base0 + iota) | lax.shift_left(RROWS + iota, 20)
            for q in range(4):
                mask = iota < (npad - q * 16)
                plsc.store_scatter(st2, [cnt_f + q * 16 + iota], fill,
                                   mask=mask)
            padded = cnt_f + npad
            cvec = jnp.where(iota == f, padded, cvec)
            pltpu.sync_copy(
                st2, bins_hbm.at[pl.ds((rng_t * NW + wid) * BCAP, BCAP)])
        cntv[...] = cvec
        pltpu.sync_copy(cntv, pcnt_hbm.at[pl.ds((wid * 8 + c) * 16, 16)])


_bin = pl.kernel(
    _bin_body,
    out_type=(
        jax.ShapeDtypeStruct((NRANGE * NW * BCAP,), jnp.int32),
        jax.ShapeDtypeStruct((NW * 8 * 16,), jnp.int32),
    ),
    mesh=plsc.VectorSubcoreMesh(core_axis_name="c", subcore_axis_name="s"),
    scratch_types=[
        pltpu.VMEM((_BW,), jnp.int32),
        pltpu.VMEM((8, _SCAP), jnp.int32),
        pltpu.VMEM((8, _SCAP), jnp.int32),
        pltpu.VMEM((BCAP,), jnp.int32),
        pltpu.VMEM((16,), jnp.int32),
        pltpu.SemaphoreType.DMA,
    ],
    compiler_params=pltpu.CompilerParams(needs_layout_passes=False),
)


_WIN = 64


def _scatter_body(msg_hbm, bins_hbm, pcnt_hbm, out_hbm,
                  ids, lbuf, cntv, rows, acc, sem):
    wid = lax.axis_index("s") * NC + lax.axis_index("c")
    iota = lax.iota(jnp.int32, 16)

    for q in range(2):
        rng = wid * 2 + q
        c = lax.shift_right_logical(rng, 3)
        f = rng & 7

        def zro(j, _):
            for ch in range(8):
                acc[j, pl.ds(ch * 16, 16)] = jnp.zeros((16,), jnp.float32)
            return 0

        lax.fori_loop(0, ACC_ROWS, zro, 0)

        def wloop(w, _):
            pltpu.sync_copy(pcnt_hbm.at[pl.ds((w * 8 + c) * 16, 16)], cntv)
            cnt = jnp.max(jnp.where(iota == f, cntv[...], 0))
            nwin = lax.shift_right_logical(cnt, 6)
            base = (rng * NW + w) * BCAP

            def wina(j, _):
                pltpu.sync_copy(bins_hbm.at[pl.ds(base + j * _WIN, _WIN)],
                                ids)
                for v in range(4):
                    pk = ids[pl.ds(v * 16, 16)]
                    lbuf[pl.ds(v * 16, 16)] = lax.shift_right_logical(pk, 20)
                    ids[pl.ds(v * 16, 16)] = pk & 0xFFFFF
                pltpu.async_copy(msg_hbm.at[ids], rows, sem).wait()

                def edge(e, _):
                    l16 = lbuf[pl.ds((e // 16) * 16, 16)]
                    loc = jnp.take(l16, jnp.full((16,), e % 16, jnp.int32))
                    for ch in range(8):
                        vals = rows[e, pl.ds(ch * 16, 16)]
                        plsc.addupdate_scatter(acc, [loc, ch * 16 + iota],
                                               vals)
                    return 0

                lax.fori_loop(0, _WIN, edge, 0)
                return 0

            lax.fori_loop(0, nwin, wina, 0)
            return 0

        lax.fori_loop(0, NW, wloop, 0)
        pltpu.sync_copy(acc, out_hbm.at[pl.ds(rng * ACC_ROWS, ACC_ROWS)])


_scatter = pl.kernel(
    _scatter_body,
    out_type=jax.ShapeDtypeStruct((NRANGE * ACC_ROWS, 128), jnp.float32),
    mesh=plsc.VectorSubcoreMesh(core_axis_name="c", subcore_axis_name="s"),
    scratch_types=[
        pltpu.VMEM((_WIN,), jnp.int32),
        pltpu.VMEM((_WIN,), jnp.int32),
        pltpu.VMEM((16,), jnp.int32),
        pltpu.VMEM((_WIN, 128), jnp.float32),
        pltpu.VMEM((ACC_ROWS, 128), jnp.float32),
        pltpu.SemaphoreType.DMA,
    ],
    compiler_params=pltpu.CompilerParams(needs_layout_passes=False),
)


@jax.jit
def kernel(node_feats, edge_feats, scalar_edge_feats, lengths, senders,
           receivers, W1, W2):
    nf128 = jnp.pad(node_feats, ((0, 0), (0, 48)))
    gth = _gather(nf128, senders, receivers)
    msg = _edge_compute(gth, edge_feats, scalar_edge_feats, lengths, W1, W2)
    bins, pcnt = _bin(receivers)
    out = _scatter(msg, bins, pcnt)
    out = out.reshape(NRANGE, ACC_ROWS, 128)[:, :RROWS, :]
    out = out.reshape(NPAD, 128)[:N_NODES]
    o1 = out[:, 16:112].reshape(N_NODES, 3, S0).transpose(0, 2, 1)
    return jnp.concatenate([out[:, :16], o1.reshape(N_NODES, 96)], axis=1)


# speedup vs baseline: 2.9074x; 1.0490x over previous
import math

import jax
import jax.numpy as jnp
import numpy as np
from jax import lax
from jax.experimental import pallas as pl
from jax.experimental.pallas import tpu as pltpu
from jax.experimental.pallas import tpu_sc as plsc

N_NODES = 50000
N_EDGES = 800000
S0 = 32
V0 = 16
MLP_IN = 73
HID = 64
AVG_NUM_NEIGHBORS = 16.0

NC = 2
NS = 16
NW = NC * NS

RROWS = 784
NRANGE = 64
NPAD = RROWS * NRANGE
ACC_ROWS = 800
BCAP = 1088
_INV = 1.0 / 784.0


_GK = 200
_EPW = N_EDGES // NW


def _gather_body(nf_hbm, snd_hbm, rcv_hbm, out_hbm,
                 sidx, ridx, srow, rrow, sem_s, sem_r):
    wid = lax.axis_index("s") * NC + lax.axis_index("c")
    base0 = wid * _EPW

    def step(i, _):
        base = base0 + i * _GK
        pltpu.sync_copy(snd_hbm.at[pl.ds(base, _GK)], sidx)
        pltpu.sync_copy(rcv_hbm.at[pl.ds(base, _GK)], ridx)
        c1 = pltpu.async_copy(nf_hbm.at[sidx], srow, sem_s)
        c2 = pltpu.async_copy(nf_hbm.at[ridx], rrow, sem_r)
        c1.wait()
        c2.wait()

        def pack(j, _):
            srow[j, pl.ds(80, 16)] = rrow[j, pl.ds(0, 16)]
            srow[j, pl.ds(96, 16)] = rrow[j, pl.ds(16, 16)]
            return 0

        lax.fori_loop(0, _GK, pack, 0)
        pltpu.sync_copy(srow, out_hbm.at[pl.ds(base, _GK)])
        return 0

    lax.fori_loop(0, _EPW // _GK, step, 0)


_gather = pl.kernel(
    _gather_body,
    out_type=jax.ShapeDtypeStruct((N_EDGES, 128), jnp.float32),
    mesh=plsc.VectorSubcoreMesh(core_axis_name="c", subcore_axis_name="s"),
    scratch_types=[
        pltpu.VMEM((_GK,), jnp.int32),
        pltpu.VMEM((_GK,), jnp.int32),
        pltpu.VMEM((_GK, 128), jnp.float32),
        pltpu.VMEM((_GK, 128), jnp.float32),
        pltpu.SemaphoreType.DMA,
        pltpu.SemaphoreType.DMA,
    ],
)


_EB = 4000

_T48 = np.zeros((3, 48), np.float32)
_T48[np.arange(48) % 3, np.arange(48)] = 1.0
_P48 = np.zeros((48, 16), np.float32)
_P48[np.arange(48), np.arange(48) // 3] = 1.0
_T96 = np.zeros((3, 96), np.float32)
_T96[np.arange(96) % 3, np.arange(96)] = 1.0
_R96 = np.zeros((32, 96), np.float32)
_R96[np.arange(96) // 3, np.arange(96)] = 1.0


def _split_dot(x, w):
    xh = x.astype(jnp.bfloat16)
    xl = (x - xh.astype(jnp.float32)).astype(jnp.bfloat16)
    wh = w.astype(jnp.bfloat16)
    wl = (w - wh.astype(jnp.float32)).astype(jnp.bfloat16)
    d = lambda a, b: jnp.dot(a, b, preferred_element_type=jnp.float32)
    return d(xh, wh) + (d(xh, wl) + d(xl, wh))


def _edge_body(gth, ef, sef, ln, w1, w2, p48, out):
    ms = gth[:, :S0]
    sv = gth[:, S0:S0 + 3 * V0]
    rs = gth[:, 80:112]
    ev = ef[:, 1:4]
    x = jnp.concatenate([ms, rs, sef[...], ln[...]], axis=1)
    h = _split_dot(x, w1[...]) * (1.0 / math.sqrt(float(MLP_IN)))
    h = h / (1.0 + jnp.exp(-h))
    mix = _split_dot(h, w2[...]) * (1.0 / math.sqrt(float(HID)))
    ev48 = jnp.concatenate([ev] * 16, axis=1)
    o0 = _split_dot(sv * ev48, p48[...])
    o0 = o0 * mix[:, :V0] * (1.0 / (math.sqrt(3.0) * AVG_NUM_NEIGHBORS))
    a = ms * mix[:, V0:] * (1.0 / AVG_NUM_NEIGHBORS)
    o1 = jnp.concatenate(
        [a * ef[:, 1:2], a * ef[:, 2:3], a * ef[:, 3:4]], axis=1)
    out[...] = jnp.concatenate(
        [o0, o1, jnp.zeros((_EB, 16), jnp.float32)], axis=1)


def _edge_compute(gth, ef, sef, ln, w1, w2):
    nb = N_EDGES // _EB
    full = lambda i: (0, 0)
    return pl.pallas_call(
        _edge_body,
        grid=(nb,),
        in_specs=[
            pl.BlockSpec((_EB, 128), lambda i: (i, 0)),
            pl.BlockSpec((_EB, 4), lambda i: (i, 0)),
            pl.BlockSpec((_EB, 8), lambda i: (i, 0)),
            pl.BlockSpec((_EB, 1), lambda i: (i, 0)),
            pl.BlockSpec((MLP_IN, HID), full),
            pl.BlockSpec((HID, 48), full),
            pl.BlockSpec((48, 16), full),
        ],
        out_specs=pl.BlockSpec((_EB, 128), lambda i: (i, 0)),
        out_shape=jax.ShapeDtypeStruct((N_EDGES, 128), jnp.float32),
        compiler_params=pltpu.CompilerParams(
            dimension_semantics=("arbitrary",),
        ),
    )(gth, ef, sef, ln, w1, w2, jnp.asarray(_P48))


_BW = 1000
_SCAP = 3920


def _bin_body(rcv_hbm, bins_hbm, pcnt_hbm, win, st_e, st_r, st2, cntv, sem):
    wid = lax.axis_index("s") * NC + lax.axis_index("c")
    base0 = wid * _EPW
    iota = lax.iota(jnp.int32, 16)
    inv = jnp.float32(_INV)

    def window_step(i, curs):
        base = base0 + i * _BW
        pltpu.sync_copy(rcv_hbm.at[pl.ds(base, _BW)], win)
        new_curs = []
        for c in range(8):
            def vstep(j, cur, c=c):
                off = jnp.where(j < 62, j * 16, 984)
                valid = (j < 62) | (iota >= 8)
                r = win[pl.ds(off, 16)]
                rng = ((r.astype(jnp.float32) + 0.5) * inv).astype(jnp.int32)
                mask = valid & (lax.shift_right_logical(rng, 3) == c)
                eid = base + off + iota
                cum = plsc.cumsum(mask.astype(jnp.int32))
                pos = cur + cum - mask.astype(jnp.int32)
                crow = jnp.full((16,), c, jnp.int32)
                plsc.store_scatter(st_r, [crow, pos], r, mask=mask)
                plsc.store_scatter(st_e, [crow, pos], eid, mask=mask)
                return cur + jnp.max(cum)

            new_curs.append(lax.fori_loop(0, 63, vstep, curs[c]))
        return tuple(new_curs)

    curs = lax.fori_loop(0, _EPW // _BW, window_step, (0,) * 8)

    for c in range(8):
        cnt_c = curs[c]
        trips = lax.shift_right_logical(cnt_c + 15, 4)
        cvec = jnp.zeros((16,), jnp.int32)
        for f in range(8):
            rng_t = c * 8 + f

            def vstep(j, cur, c=c, rng_t=rng_t):
                valid = (j * 16 + iota) < cnt_c
                r = st_r[c, pl.ds(j * 16, 16)]
                e = st_e[c, pl.ds(j * 16, 16)]
                rng = ((r.astype(jnp.float32) + 0.5) * inv).astype(jnp.int32)
                mask = valid & (rng == rng_t)
                local = r - rng * RROWS
                packed = e | lax.shift_left(local, 20)
                cum = plsc.cumsum(mask.astype(jnp.int32))
                pos = cur + cum - mask.astype(jnp.int32)
                plsc.store_scatter(st2, [pos], packed, mask=mask)
                return cur + jnp.max(cum)

            cnt_f = lax.fori_loop(0, trips, vstep, 0)
            npad = (-cnt_f) & 63
            fill = (base0 + iota) | lax.shift_left(RROWS + iota, 20)
            for q in range(4):
                mask = iota < (npad - q * 16)
                plsc.store_scatter(st2, [cnt_f + q * 16 + iota], fill,
                                   mask=mask)
            padded = cnt_f + npad
            cvec = jnp.where(iota == f, padded, cvec)
            pltpu.sync_copy(
                st2, bins_hbm.at[pl.ds((rng_t * NW + wid) * BCAP, BCAP)])
        cntv[...] = cvec
        pltpu.sync_copy(cntv, pcnt_hbm.at[pl.ds((wid * 8 + c) * 16, 16)])


_bin = pl.kernel(
    _bin_body,
    out_type=(
        jax.ShapeDtypeStruct((NRANGE * NW * BCAP,), jnp.int32),
        jax.ShapeDtypeStruct((NW * 8 * 16,), jnp.int32),
    ),
    mesh=plsc.VectorSubcoreMesh(core_axis_name="c", subcore_axis_name="s"),
    scratch_types=[
        pltpu.VMEM((_BW,), jnp.int32),
        pltpu.VMEM((8, _SCAP), jnp.int32),
        pltpu.VMEM((8, _SCAP), jnp.int32),
        pltpu.VMEM((BCAP,), jnp.int32),
        pltpu.VMEM((16,), jnp.int32),
        pltpu.SemaphoreType.DMA,
    ],
    compiler_params=pltpu.CompilerParams(needs_layout_passes=False),
)


_WIN = 64


def _scatter_body(msg_hbm, bins_hbm, pcnt_hbm, out_hbm,
                  ids_a, ids_b, lbuf_a, lbuf_b, cntv, rows_a, rows_b, acc,
                  sem_a, sem_b):
    wid = lax.axis_index("s") * NC + lax.axis_index("c")
    iota = lax.iota(jnp.int32, 16)

    def load_unpack(base, j, ids, lbuf):
        pltpu.sync_copy(bins_hbm.at[pl.ds(base + j * _WIN, _WIN)], ids)
        for v in range(4):
            pk = ids[pl.ds(v * 16, 16)]
            lbuf[pl.ds(v * 16, 16)] = lax.shift_right_logical(pk, 20)
            ids[pl.ds(v * 16, 16)] = pk & 0xFFFFF

    def accum(acc, rows, lbuf):
        def edge(e, _):
            l16 = lbuf[pl.ds((e // 16) * 16, 16)]
            loc = jnp.take(l16, jnp.full((16,), e % 16, jnp.int32))
            for ch in range(8):
                plsc.addupdate_scatter(acc, [loc, ch * 16 + iota],
                                       rows[e, pl.ds(ch * 16, 16)])
            return 0

        lax.fori_loop(0, _WIN, edge, 0)

    for q in range(2):
        rng = wid * 2 + q
        c = lax.shift_right_logical(rng, 3)
        f = rng & 7

        def zro(j, _):
            for ch in range(8):
                acc[j, pl.ds(ch * 16, 16)] = jnp.zeros((16,), jnp.float32)
            return 0

        lax.fori_loop(0, ACC_ROWS, zro, 0)

        def wloop(w, _):
            pltpu.sync_copy(pcnt_hbm.at[pl.ds((w * 8 + c) * 16, 16)], cntv)
            cnt = jnp.max(jnp.where(iota == f, cntv[...], 0))
            nwin = lax.shift_right_logical(cnt, 6)
            base = (rng * NW + w) * BCAP

            def pair(k, _):
                load_unpack(base, k * 2, ids_a, lbuf_a)
                ca = pltpu.async_copy(msg_hbm.at[ids_a], rows_a, sem_a)
                load_unpack(base, k * 2 + 1, ids_b, lbuf_b)
                cb = pltpu.async_copy(msg_hbm.at[ids_b], rows_b, sem_b)
                ca.wait()
                accum(acc, rows_a, lbuf_a)
                cb.wait()
                accum(acc, rows_b, lbuf_b)
                return 0

            lax.fori_loop(0, lax.shift_right_logical(nwin, 1), pair, 0)

            @pl.when((nwin & 1) == 1)
            def _():
                load_unpack(base, nwin - 1, ids_a, lbuf_a)
                pltpu.async_copy(msg_hbm.at[ids_a], rows_a, sem_a).wait()
                accum(acc, rows_a, lbuf_a)

            return 0

        lax.fori_loop(0, NW, wloop, 0)
        pltpu.sync_copy(acc, out_hbm.at[pl.ds(rng * ACC_ROWS, ACC_ROWS)])


_scatter = pl.kernel(
    _scatter_body,
    out_type=jax.ShapeDtypeStruct((NRANGE * ACC_ROWS, 128), jnp.float32),
    mesh=plsc.VectorSubcoreMesh(core_axis_name="c", subcore_axis_name="s"),
    scratch_types=[
        pltpu.VMEM((_WIN,), jnp.int32),
        pltpu.VMEM((_WIN,), jnp.int32),
        pltpu.VMEM((_WIN,), jnp.int32),
        pltpu.VMEM((_WIN,), jnp.int32),
        pltpu.VMEM((16,), jnp.int32),
        pltpu.VMEM((_WIN, 128), jnp.float32),
        pltpu.VMEM((_WIN, 128), jnp.float32),
        pltpu.VMEM((ACC_ROWS, 128), jnp.float32),
        pltpu.SemaphoreType.DMA,
        pltpu.SemaphoreType.DMA,
    ],
    compiler_params=pltpu.CompilerParams(needs_layout_passes=False),
)


@jax.jit
def kernel(node_feats, edge_feats, scalar_edge_feats, lengths, senders,
           receivers, W1, W2):
    nf128 = jnp.pad(node_feats, ((0, 0), (0, 48)))
    gth = _gather(nf128, senders, receivers)
    msg = _edge_compute(gth, edge_feats, scalar_edge_feats, lengths, W1, W2)
    bins, pcnt = _bin(receivers)
    out = _scatter(msg, bins, pcnt)
    out = out.reshape(NRANGE, ACC_ROWS, 128)[:, :RROWS, :]
    out = out.reshape(NPAD, 128)[:N_NODES]
    o1 = out[:, 16:112].reshape(N_NODES, 3, S0).transpose(0, 2, 1)
    return jnp.concatenate([out[:, :16], o1.reshape(N_NODES, 96)], axis=1)
